# baseline (speedup 1.0000x reference)
import functools

import jax
import jax.numpy as jnp
from jax.experimental import pallas as pl

_K = 32


def _knn_body(q_ref, pt_ref, q2_ref, p2_ref, e_ref, o_ref):
    dot = jax.lax.dot_general(q_ref[...], pt_ref[...],
                              (((1,), (0,)), ((), ())),
                              preferred_element_type=jnp.float32)
    d2 = jnp.maximum((q2_ref[...] + p2_ref[...]) - 2.0 * dot, 0.0)
    bits = jax.lax.bitcast_convert_type(d2, jnp.int32)
    rows = d2.shape[0]

    ones_cnt = jnp.ones((d2.shape[1], 8), jnp.bfloat16)
    kf = jnp.float32(_K)

    def probe(i, cand):
        sh = 30 - i
        t = cand | ((1 << sh) - 1)
        m = (bits <= t).astype(jnp.bfloat16)
        cnt = jax.lax.dot_general(m, ones_cnt, (((1,), (0,)), ((), ())),
                                  preferred_element_type=jnp.float32)[:, :1]
        return jnp.where(cnt >= kf, cand, cand | (1 << sh))

    v32 = jax.lax.fori_loop(0, 31, probe,
                            jnp.zeros((rows, 1), jnp.int32))

    less = bits < v32
    tie = bits == v32
    n_le = jnp.sum((bits <= v32).astype(jnp.int32), axis=1, keepdims=True)

    col = jax.lax.broadcasted_iota(jnp.int32, tie.shape, 1)
    jbits = (tie.shape[1] - 1).bit_length()
    n_cols = tie.shape[1]

    def tie_search(_):
        n_less = jnp.sum(less.astype(jnp.int32), axis=1, keepdims=True)
        extra = _K - n_less

        def probe_j(i, cand):
            sh = jbits - 1 - i
            t = cand | ((1 << sh) - 1)
            cnt = jnp.sum((tie & (col <= t)).astype(jnp.int32),
                          axis=1, keepdims=True)
            return jnp.where(cnt >= extra, cand, cand | (1 << sh))

        return jax.lax.fori_loop(0, jbits, probe_j,
                                 jnp.zeros((rows, 1), jnp.int32))

    jsel = jax.lax.cond(jnp.max(n_le) > _K, tie_search,
                        lambda _: jnp.full((rows, 1), n_cols, jnp.int32),
                        operand=None)
    sel = less | (tie & (col <= jsel))

    w = jnp.where(sel, 1.0 / (jnp.sqrt(d2 + 1e-12) + 1e-8), 0.0)
    acc = jax.lax.dot_general(w, e_ref[...], (((1,), (0,)), ((), ())),
                              precision=jax.lax.Precision.HIGHEST,
                              preferred_element_type=jnp.float32)
    c = o_ref.shape[1]
    o_ref[...] = acc[:, :c] / acc[:, c:c + 1]


def _run(qpad, pt, q2, p2, feats, rows_per_block):
    q_total = qpad.shape[0]
    n = pt.shape[1]
    ce = feats.shape[1]
    c = ce - 1
    grid = q_total // rows_per_block
    return pl.pallas_call(
        _knn_body,
        grid=(grid,),
        in_specs=[
            pl.BlockSpec((rows_per_block, 8), lambda i: (i, 0)),
            pl.BlockSpec((8, n), lambda i: (0, 0)),
            pl.BlockSpec((rows_per_block, 1), lambda i: (i, 0)),
            pl.BlockSpec((1, n), lambda i: (0, 0)),
            pl.BlockSpec((n, ce), lambda i: (0, 0)),
        ],
        out_specs=pl.BlockSpec((rows_per_block, c), lambda i: (i, 0)),
        out_shape=jax.ShapeDtypeStruct((q_total, c), jnp.float32),
    )(qpad, pt, q2, p2, feats)


def kernel(queries, xyz, points_embeding):
    q_total = queries.shape[0]
    n = xyz.shape[0]
    q2 = jnp.sum(queries * queries, axis=-1, keepdims=True)
    p2 = jnp.sum(xyz * xyz, axis=-1)[None, :]
    qpad = jnp.concatenate(
        [queries, jnp.zeros((q_total, 5), jnp.float32)], axis=1)
    pt = jnp.concatenate(
        [xyz, jnp.zeros((n, 5), jnp.float32)], axis=1).T
    feats = jnp.concatenate(
        [points_embeding[0], jnp.ones((n, 1), jnp.float32)], axis=1)
    return _run(qpad, pt, q2, p2, feats, rows_per_block=256)


# speedup vs baseline: 1.3292x; 1.3292x over previous
import functools

import jax
import jax.numpy as jnp
from jax.experimental import pallas as pl

_K = 32


def _knn_body(q_ref, pt_ref, q2_ref, p2_ref, ehi_ref, elo_ref, o_ref):
    dot = jax.lax.dot_general(q_ref[...], pt_ref[...],
                              (((1,), (0,)), ((), ())),
                              preferred_element_type=jnp.float32)
    d2 = jnp.maximum((q2_ref[...] + p2_ref[...]) - 2.0 * dot, 0.0)
    bits = jax.lax.bitcast_convert_type(d2, jnp.int32)
    rows = d2.shape[0]

    def probe(i, cand):
        sh = 30 - i
        t = cand | ((1 << sh) - 1)
        cnt = jnp.sum((bits <= t).astype(jnp.int32), axis=1, keepdims=True)
        return jnp.where(cnt >= _K, cand, cand | (1 << sh))

    v32 = jax.lax.fori_loop(0, 31, probe,
                            jnp.zeros((rows, 1), jnp.int32))

    less = bits < v32
    tie = bits == v32
    n_le = jnp.sum((bits <= v32).astype(jnp.int32), axis=1, keepdims=True)

    col = jax.lax.broadcasted_iota(jnp.int32, tie.shape, 1)
    jbits = (tie.shape[1] - 1).bit_length()
    n_cols = tie.shape[1]

    def tie_search(_):
        n_less = jnp.sum(less.astype(jnp.int32), axis=1, keepdims=True)
        extra = _K - n_less

        def probe_j(i, cand):
            sh = jbits - 1 - i
            t = cand | ((1 << sh) - 1)
            cnt = jnp.sum((tie & (col <= t)).astype(jnp.int32),
                          axis=1, keepdims=True)
            return jnp.where(cnt >= extra, cand, cand | (1 << sh))

        return jax.lax.fori_loop(0, jbits, probe_j,
                                 jnp.zeros((rows, 1), jnp.int32))

    jsel = jax.lax.cond(jnp.max(n_le) > _K, tie_search,
                        lambda _: jnp.full((rows, 1), n_cols, jnp.int32),
                        operand=None)
    sel = less | (tie & (col <= jsel))

    w = jnp.where(sel, 1.0 / (jnp.sqrt(d2 + 1e-12) + 1e-8), 0.0)
    w_hi = w.astype(jnp.bfloat16)
    w_lo = (w - w_hi.astype(jnp.float32)).astype(jnp.bfloat16)
    dims = (((1,), (0,)), ((), ()))
    acc = (jax.lax.dot_general(w_hi, ehi_ref[...], dims,
                               preferred_element_type=jnp.float32)
           + jax.lax.dot_general(w_hi, elo_ref[...], dims,
                                 preferred_element_type=jnp.float32)
           + jax.lax.dot_general(w_lo, ehi_ref[...], dims,
                                 preferred_element_type=jnp.float32))
    c = o_ref.shape[1]
    o_ref[...] = acc[:, :c] / acc[:, c:c + 1]


def _run(qpad, pt, q2, p2, e_hi, e_lo, rows_per_block):
    q_total = qpad.shape[0]
    n = pt.shape[1]
    ce = e_hi.shape[1]
    c = ce - 1
    grid = q_total // rows_per_block
    return pl.pallas_call(
        _knn_body,
        grid=(grid,),
        in_specs=[
            pl.BlockSpec((rows_per_block, 8), lambda i: (i, 0)),
            pl.BlockSpec((8, n), lambda i: (0, 0)),
            pl.BlockSpec((rows_per_block, 1), lambda i: (i, 0)),
            pl.BlockSpec((1, n), lambda i: (0, 0)),
            pl.BlockSpec((n, ce), lambda i: (0, 0)),
            pl.BlockSpec((n, ce), lambda i: (0, 0)),
        ],
        out_specs=pl.BlockSpec((rows_per_block, c), lambda i: (i, 0)),
        out_shape=jax.ShapeDtypeStruct((q_total, c), jnp.float32),
    )(qpad, pt, q2, p2, e_hi, e_lo)


def kernel(queries, xyz, points_embeding):
    q_total = queries.shape[0]
    n = xyz.shape[0]
    q2 = jnp.sum(queries * queries, axis=-1, keepdims=True)
    p2 = jnp.sum(xyz * xyz, axis=-1)[None, :]
    qpad = jnp.concatenate(
        [queries, jnp.zeros((q_total, 5), jnp.float32)], axis=1)
    pt = jnp.concatenate(
        [xyz, jnp.zeros((n, 5), jnp.float32)], axis=1).T
    feats = jnp.concatenate(
        [points_embeding[0], jnp.ones((n, 1), jnp.float32)], axis=1)
    e_hi = feats.astype(jnp.bfloat16)
    e_lo = (feats - e_hi.astype(jnp.float32)).astype(jnp.bfloat16)
    return _run(qpad, pt, q2, p2, e_hi, e_lo, rows_per_block=256)


# speedup vs baseline: 1.3445x; 1.0115x over previous
import functools

import jax
import jax.numpy as jnp
from jax.experimental import pallas as pl

_K = 32


def _knn_body(q_ref, pt_ref, q2_ref, p2_ref, ehi_ref, elo_ref, o_ref):
    dot = jax.lax.dot_general(q_ref[...], pt_ref[...],
                              (((1,), (0,)), ((), ())),
                              preferred_element_type=jnp.float32)
    d2 = jnp.maximum((q2_ref[...] + p2_ref[...]) - 2.0 * dot, 0.0)
    bits = jax.lax.bitcast_convert_type(d2, jnp.int32)
    rows = d2.shape[0]

    def probe(i, cand):
        sh = 30 - i
        t = cand | ((1 << sh) - 1)
        cnt = jnp.sum((bits <= t).astype(jnp.int32), axis=1, keepdims=True)
        return jnp.where(cnt >= _K, cand, cand | (1 << sh))

    v32 = jax.lax.fori_loop(0, 31, probe,
                            jnp.zeros((rows, 1), jnp.int32))

    less = bits < v32
    tie = bits == v32
    n_le = jnp.sum((bits <= v32).astype(jnp.int32), axis=1, keepdims=True)

    col = jax.lax.broadcasted_iota(jnp.int32, tie.shape, 1)
    jbits = (tie.shape[1] - 1).bit_length()
    n_cols = tie.shape[1]

    def tie_search(_):
        n_less = jnp.sum(less.astype(jnp.int32), axis=1, keepdims=True)
        extra = _K - n_less

        def probe_j(i, cand):
            sh = jbits - 1 - i
            t = cand | ((1 << sh) - 1)
            cnt = jnp.sum((tie & (col <= t)).astype(jnp.int32),
                          axis=1, keepdims=True)
            return jnp.where(cnt >= extra, cand, cand | (1 << sh))

        return jax.lax.fori_loop(0, jbits, probe_j,
                                 jnp.zeros((rows, 1), jnp.int32))

    jsel = jax.lax.cond(jnp.max(n_le) > _K, tie_search,
                        lambda _: jnp.full((rows, 1), n_cols, jnp.int32),
                        operand=None)
    sel = less | (tie & (col <= jsel))

    w = jnp.where(sel, 1.0 / (jnp.sqrt(d2 + 1e-12) + 1e-8), 0.0)
    w_hi = w.astype(jnp.bfloat16)
    w_lo = (w - w_hi.astype(jnp.float32)).astype(jnp.bfloat16)
    dims = (((1,), (0,)), ((), ()))
    acc = (jax.lax.dot_general(w_hi, ehi_ref[...], dims,
                               preferred_element_type=jnp.float32)
           + jax.lax.dot_general(w_hi, elo_ref[...], dims,
                                 preferred_element_type=jnp.float32)
           + jax.lax.dot_general(w_lo, ehi_ref[...], dims,
                                 preferred_element_type=jnp.float32))
    c = o_ref.shape[1]
    o_ref[...] = acc[:, :c] / acc[:, c:c + 1]


def _run(qpad, pt, q2, p2, e_hi, e_lo, rows_per_block):
    q_total = qpad.shape[0]
    n = pt.shape[1]
    ce = e_hi.shape[1]
    c = ce - 1
    grid = q_total // rows_per_block
    return pl.pallas_call(
        _knn_body,
        grid=(grid,),
        in_specs=[
            pl.BlockSpec((rows_per_block, 8), lambda i: (i, 0)),
            pl.BlockSpec((8, n), lambda i: (0, 0)),
            pl.BlockSpec((rows_per_block, 1), lambda i: (i, 0)),
            pl.BlockSpec((1, n), lambda i: (0, 0)),
            pl.BlockSpec((n, ce), lambda i: (0, 0)),
            pl.BlockSpec((n, ce), lambda i: (0, 0)),
        ],
        out_specs=pl.BlockSpec((rows_per_block, c), lambda i: (i, 0)),
        out_shape=jax.ShapeDtypeStruct((q_total, c), jnp.float32),
    )(qpad, pt, q2, p2, e_hi, e_lo)


def kernel(queries, xyz, points_embeding):
    q_total = queries.shape[0]
    n = xyz.shape[0]
    q2 = jnp.sum(queries * queries, axis=-1, keepdims=True)
    p2 = jnp.sum(xyz * xyz, axis=-1)[None, :]
    qpad = jnp.concatenate(
        [queries, jnp.zeros((q_total, 5), jnp.float32)], axis=1)
    pt = jnp.concatenate(
        [xyz, jnp.zeros((n, 5), jnp.float32)], axis=1).T
    feats = jnp.concatenate(
        [points_embeding[0], jnp.ones((n, 1), jnp.float32)], axis=1)
    e_hi = feats.astype(jnp.bfloat16)
    e_lo = (feats - e_hi.astype(jnp.float32)).astype(jnp.bfloat16)
    return _run(qpad, pt, q2, p2, e_hi, e_lo, rows_per_block=512)
